# baseline (speedup 1.0000x reference)
import functools

import jax
import jax.numpy as jnp
from jax import lax
from jax.experimental import pallas as pl
from jax.experimental.pallas import tpu as pltpu
from jax.experimental.pallas import tpu_sc as plsc

_NC = 2
_NS = 16
_N_PAD = 10240
_NH = _N_PAD // 2
_BLK = 640
_NBLK = _NH // _BLK
_K = 32
_C = 128
_W2 = _C // 2
_WPT = _W2 // _NS


def _rne_bf16_bits(h):
    u = lax.bitcast_convert_type(h, jnp.int32)
    return (u + 0x7FFF + ((u >> 16) & 1)) >> 16


def _pre_body(x_ref, wp_ref, i_ref, o_ref, o2_ref):
    idx = i_ref[...].reshape(2, _NBLK, _BLK, _K)
    o2_ref[...] = lax.transpose(idx, (0, 1, 3, 2))
    x = x_ref[...]
    hl = jnp.maximum(
        lax.dot_general(wp_ref[pl.ds(0, _W2), :], x, (((1,), (0,)), ((), ())),
                        preferred_element_type=jnp.float32), 0.0)
    hh = jnp.maximum(
        lax.dot_general(wp_ref[pl.ds(_W2, _W2), :], x,
                        (((1,), (0,)), ((), ())),
                        preferred_element_type=jnp.float32), 0.0)
    o_ref[...] = (_rne_bf16_bits(hh) << 16) | _rne_bf16_bits(hl)


def _post_body(x_ref, ap_ref, wn_ref, b_ref, o_ref):
    wx = wn_ref[:, pl.ds(0, _C)]
    wal = wn_ref[:, pl.ds(_C, _W2)]
    wah = wn_ref[:, pl.ds(_C + _W2, _W2)]
    for h in range(2):
        a = ap_ref[h]
        lof = lax.bitcast_convert_type(a << 16, jnp.float32)
        hif = lax.bitcast_convert_type(a & jnp.int32(-65536), jnp.float32)
        t = lax.dot_general(wx, x_ref[:, pl.ds(h * _NH, _NH)],
                            (((1,), (0,)), ((), ())),
                            preferred_element_type=jnp.float32)
        t = t + lax.dot_general(wal, lof, (((1,), (0,)), ((), ())),
                                preferred_element_type=jnp.float32)
        t = t + lax.dot_general(wah, hif, (((1,), (0,)), ((), ())),
                                preferred_element_type=jnp.float32)
        t = jnp.maximum(t, 0.0) + b_ref[...]
        nrm = jnp.sqrt(jnp.sum(t * t, axis=0, keepdims=True))
        o_ref[:, pl.ds(h * _NH, _NH)] = t / jnp.maximum(nrm, 1e-12)


def _sc_gather_max(tbl, idx4):
    mesh = plsc.VectorSubcoreMesh(core_axis_name="c", subcore_axis_name="s")

    @functools.partial(
        pl.kernel, mesh=mesh,
        compiler_params=pltpu.CompilerParams(needs_layout_passes=False),
        out_type=jax.ShapeDtypeStruct((2, _W2, _NH), jnp.int32),
        scratch_types=[
            pltpu.VMEM((_WPT, _N_PAD), jnp.int32),
            pltpu.VMEM((2, _K, _BLK), jnp.int32),
            pltpu.VMEM((_WPT, _NH), jnp.int32),
            pltpu.SemaphoreType.DMA,
            pltpu.SemaphoreType.DMA,
        ],
    )
    def k(tbl_hbm, idx_hbm, out_hbm, tbl_v, idx_v, out_v, s0, s1):
        nh = lax.axis_index("c")
        wq = lax.axis_index("s")
        pltpu.sync_copy(tbl_hbm.at[pl.ds(wq * _WPT, _WPT)], tbl_v)
        sems = (s0, s1)
        for b in range(2):
            pltpu.async_copy(idx_hbm.at[nh, b], idx_v.at[b], sems[b])

        def blk_body(b2, carry):
            for b in range(2):
                blk = b2 * 2 + b
                pltpu.make_async_copy(
                    idx_hbm.at[nh, blk], idx_v.at[b], sems[b]).wait()

                def grp_body(g, carry2, b=b, blk=blk):
                    accs = [None] * _WPT
                    for kk in range(_K):
                        vidx = idx_v[b, kk, pl.ds(g * 16, 16)]
                        for w in range(_WPT):
                            wvec = jnp.full((16,), w, jnp.int32)
                            v = plsc.bitcast(
                                plsc.load_gather(tbl_v, [wvec, vidx]),
                                jnp.bfloat16)
                            accs[w] = v if kk == 0 else jnp.maximum(accs[w], v)
                    for w in range(_WPT):
                        out_v[w, pl.ds(blk * _BLK + g * 16, 16)] = (
                            plsc.bitcast(accs[w], jnp.int32))
                    return carry2

                lax.fori_loop(0, _BLK // 16, grp_body, 0)

                @pl.when(blk + 2 < _NBLK)
                def _(b=b, blk=blk):
                    pltpu.async_copy(
                        idx_hbm.at[nh, blk + 2], idx_v.at[b], sems[b])
            return carry

        lax.fori_loop(0, _NBLK // 2, blk_body, 0)
        pltpu.sync_copy(out_v, out_hbm.at[nh, pl.ds(wq * _WPT, _WPT)])

    return k(tbl, idx4)


def kernel(x, edge_index, W_pre, W_nn, bias):
    B, C_in, N, _ = x.shape
    C_out = W_pre.shape[0]
    assert (B, C_in, C_out, edge_index.shape[-1]) == (1, _C, _C, _K)

    X = x[0, :, :, 0]
    idx = edge_index[0, 0]
    pad = _N_PAD - N
    Xp = jnp.pad(X, ((0, 0), (0, pad)))
    idxp = jnp.pad(idx, ((0, pad), (0, 0)))

    tbl, idx4 = pl.pallas_call(
        _pre_body,
        out_shape=(jax.ShapeDtypeStruct((_W2, _N_PAD), jnp.int32),
                   jax.ShapeDtypeStruct((2, _NBLK, _K, _BLK), jnp.int32)),
    )(Xp, W_pre, idxp)

    aggr_p = _sc_gather_max(tbl, idx4)

    o = pl.pallas_call(
        _post_body,
        out_shape=jax.ShapeDtypeStruct((C_out, _N_PAD), jnp.float32),
    )(Xp, aggr_p, W_nn, bias.reshape(C_out, 1))

    return o[:, :N].reshape(1, C_out, N, 1)


# speedup vs baseline: 1.0924x; 1.0924x over previous
import functools

import jax
import jax.numpy as jnp
from jax import lax
from jax.experimental import pallas as pl
from jax.experimental.pallas import tpu as pltpu
from jax.experimental.pallas import tpu_sc as plsc

_NC = 2
_NS = 16
_N_PAD = 10240
_NH = _N_PAD // 2
_BLK = 640
_NBLK = _NH // _BLK
_K = 32
_C = 128
_W2 = _C // 2
_WPT = _W2 // _NS


def _rne_bf16_bits(h):
    u = lax.bitcast_convert_type(h, jnp.int32)
    return (u + 0x7FFF + ((u >> 16) & 1)) >> 16


def _pre_body(x_ref, wp_ref, o_ref):
    x = x_ref[...]
    hl = jnp.maximum(
        lax.dot_general(wp_ref[pl.ds(0, _W2), :], x, (((1,), (0,)), ((), ())),
                        preferred_element_type=jnp.float32), 0.0)
    hh = jnp.maximum(
        lax.dot_general(wp_ref[pl.ds(_W2, _W2), :], x,
                        (((1,), (0,)), ((), ())),
                        preferred_element_type=jnp.float32), 0.0)
    o_ref[...] = (_rne_bf16_bits(hh) << 16) | _rne_bf16_bits(hl)


def _post_body(x_ref, ap_ref, wn_ref, b_ref, o_ref):
    wx = wn_ref[:, pl.ds(0, _C)]
    wal = wn_ref[:, pl.ds(_C, _W2)]
    wah = wn_ref[:, pl.ds(_C + _W2, _W2)]
    for h in range(2):
        a = ap_ref[h]
        lof = lax.bitcast_convert_type(a << 16, jnp.float32)
        hif = lax.bitcast_convert_type(a & jnp.int32(-65536), jnp.float32)
        t = lax.dot_general(wx, x_ref[:, pl.ds(h * _NH, _NH)],
                            (((1,), (0,)), ((), ())),
                            preferred_element_type=jnp.float32)
        t = t + lax.dot_general(wal, lof, (((1,), (0,)), ((), ())),
                                preferred_element_type=jnp.float32)
        t = t + lax.dot_general(wah, hif, (((1,), (0,)), ((), ())),
                                preferred_element_type=jnp.float32)
        t = jnp.maximum(t, 0.0) + b_ref[...]
        nrm = jnp.sqrt(jnp.sum(t * t, axis=0, keepdims=True))
        o_ref[:, pl.ds(h * _NH, _NH)] = t / jnp.maximum(nrm, 1e-12)


def _sc_gather_max(tbl, idx4):
    mesh = plsc.VectorSubcoreMesh(core_axis_name="c", subcore_axis_name="s")

    @functools.partial(
        pl.kernel, mesh=mesh,
        compiler_params=pltpu.CompilerParams(needs_layout_passes=False),
        out_type=jax.ShapeDtypeStruct((2, _W2, _NH), jnp.int32),
        scratch_types=[
            pltpu.VMEM((_WPT, _N_PAD), jnp.int32),
            pltpu.VMEM((2, _K, _BLK), jnp.int32),
            pltpu.VMEM((_WPT, _NH), jnp.int32),
            pltpu.SemaphoreType.DMA,
            pltpu.SemaphoreType.DMA,
        ],
    )
    def k(tbl_hbm, idx_hbm, out_hbm, tbl_v, idx_v, out_v, s0, s1):
        nh = lax.axis_index("c")
        wq = lax.axis_index("s")
        pltpu.sync_copy(tbl_hbm.at[pl.ds(wq * _WPT, _WPT)], tbl_v)
        sems = (s0, s1)
        for b in range(2):
            pltpu.async_copy(idx_hbm.at[nh, b], idx_v.at[b], sems[b])

        def blk_body(b2, carry):
            for b in range(2):
                blk = b2 * 2 + b
                pltpu.make_async_copy(
                    idx_hbm.at[nh, blk], idx_v.at[b], sems[b]).wait()

                def grp_body(g, carry2, b=b, blk=blk):
                    accs = [None] * _WPT
                    for kk in range(_K):
                        vidx = idx_v[b, kk, pl.ds(g * 16, 16)]
                        for w in range(_WPT):
                            wvec = jnp.full((16,), w, jnp.int32)
                            v = plsc.bitcast(
                                plsc.load_gather(tbl_v, [wvec, vidx]),
                                jnp.bfloat16)
                            accs[w] = v if kk == 0 else jnp.maximum(accs[w], v)
                    for w in range(_WPT):
                        out_v[w, pl.ds(blk * _BLK + g * 16, 16)] = (
                            plsc.bitcast(accs[w], jnp.int32))
                    return carry2

                lax.fori_loop(0, _BLK // 16, grp_body, 0)

                @pl.when(blk + 2 < _NBLK)
                def _(b=b, blk=blk):
                    pltpu.async_copy(
                        idx_hbm.at[nh, blk + 2], idx_v.at[b], sems[b])
            return carry

        lax.fori_loop(0, _NBLK // 2, blk_body, 0)
        pltpu.sync_copy(out_v, out_hbm.at[nh, pl.ds(wq * _WPT, _WPT)])

    return k(tbl, idx4)


def kernel(x, edge_index, W_pre, W_nn, bias):
    B, C_in, N, _ = x.shape
    C_out = W_pre.shape[0]
    assert (B, C_in, C_out, edge_index.shape[-1]) == (1, _C, _C, _K)

    X = x[0, :, :, 0]
    idx = edge_index[0, 0]
    pad = _N_PAD - N
    Xp = jnp.pad(X, ((0, 0), (0, pad)))
    idxp = jnp.pad(idx, ((0, pad), (0, 0)))
    idx4 = idxp.reshape(2, _NBLK, _BLK, _K).transpose(0, 1, 3, 2)

    tbl = pl.pallas_call(
        _pre_body,
        out_shape=jax.ShapeDtypeStruct((_W2, _N_PAD), jnp.int32),
    )(Xp, W_pre)

    aggr_p = _sc_gather_max(tbl, idx4)

    o = pl.pallas_call(
        _post_body,
        out_shape=jax.ShapeDtypeStruct((C_out, _N_PAD), jnp.float32),
    )(Xp, aggr_p, W_nn, bias.reshape(C_out, 1))

    return o[:, :N].reshape(1, C_out, N, 1)
